# Initial kernel scaffold; baseline (speedup 1.0000x reference)
import jax
import jax.numpy as jnp
from jax.experimental import pallas as pl


def kernel(x, edge_index, edge_attr, batch, params):
    raise NotImplementedError("write your pallas kernel here")


# speedup vs baseline: 5.0640x; 5.0640x over previous
import functools

import jax
import jax.numpy as jnp
from jax.experimental import pallas as pl

HEADS = 4
EMB = 256
LAYERS = 3


def _mm_body(a_ref, b_ref, o_ref):
    o_ref[...] = jnp.dot(a_ref[...], b_ref[...],
                         preferred_element_type=jnp.float32)


def _matmul(a, b, bm):
    m, k = a.shape
    _, n = b.shape
    return pl.pallas_call(
        _mm_body,
        grid=(m // bm,),
        in_specs=[
            pl.BlockSpec((bm, k), lambda i: (i, 0)),
            pl.BlockSpec((k, n), lambda i: (0, 0)),
        ],
        out_specs=pl.BlockSpec((bm, n), lambda i: (i, 0)),
        out_shape=jax.ShapeDtypeStruct((m, n), jnp.float32),
    )(a, b)


def kernel(x, edge_index, edge_attr, batch, params):
    n = x.shape[0]
    e = edge_index.shape[1]
    b_graphs = 64

    loops = jnp.arange(n, dtype=edge_index.dtype)
    src = jnp.concatenate([edge_index[0], loops])
    dst = jnp.concatenate([edge_index[1], loops])
    ea_mean = jnp.mean(edge_attr, 0)

    h = jax.nn.relu(_matmul(x, params['W0'], bm=1000) + params['b0'])

    outs = []
    for lp in params['layers']:
        r = h
        w3 = lp['W'].reshape(EMB, HEADS, EMB)
        w_src = jnp.einsum('khe,he->kh', w3, lp['att_src'])
        w_dst = jnp.einsum('khe,he->kh', w3, lp['att_dst'])
        we3 = lp['We'].reshape(-1, HEADS, EMB)
        we_att = jnp.einsum('khe,he->kh', we3, lp['att_e'])

        xh = _matmul(h, lp['W'], bm=1000).reshape(n, HEADS, EMB)
        s_src = h @ w_src
        s_dst = h @ w_dst
        s_e_real = edge_attr @ we_att
        s_e_loop = ea_mean @ we_att
        s_e = jnp.concatenate(
            [s_e_real, jnp.broadcast_to(s_e_loop, (n, HEADS))], 0)

        a = s_src[src] + s_dst[dst] + s_e
        a = jax.nn.leaky_relu(a, 0.2)
        ubound = jax.nn.leaky_relu(
            jnp.max(s_src, 0) + jnp.max(s_dst, 0) + jnp.max(s_e, 0), 0.2)
        ex = jnp.exp(a - ubound)
        den = jax.ops.segment_sum(ex, dst, num_segments=n)
        alpha = ex / den[dst]
        msg = xh[src] * alpha[..., None]
        out = jax.ops.segment_sum(
            msg.reshape(e + n, HEADS * EMB), dst, num_segments=n)
        out = out.reshape(n, HEADS, EMB).mean(axis=1) + lp['bias']

        m = jnp.mean(out, 0)
        v = jnp.var(out, 0)
        out = (out - m) / jnp.sqrt(v + 1e-5) * lp['gamma'] + lp['beta']
        h = r + jax.nn.relu(out)
        outs.append(h)

    onehot = (batch[:, None] == jnp.arange(b_graphs)[None, :]).astype(jnp.float32)
    pooled = [onehot.T @ o for o in outs]
    zs = jnp.concatenate(pooled, axis=1)
    gates = jax.nn.softmax(zs @ params['Wg'] + params['bg'], axis=1)
    zt = jnp.stack(pooled, axis=1)
    z = jnp.sum(zt * gates[..., None], axis=1)
    return (z, outs[-1])


# speedup vs baseline: 7.9109x; 1.5622x over previous
import functools

import jax
import jax.numpy as jnp
from jax import lax
from jax.experimental import pallas as pl
from jax.experimental.pallas import tpu as pltpu
from jax.experimental.pallas import tpu_sc as plsc

HEADS = 4
EMB = 256
N_NODES = 10000
N_PAD = 10240
E_REAL = 160000
E_TOT = 170000
E_PAD = 172032
EPW = 5376
K = 128
NCHUNK = EPW // K
NC = 2
NS = 16
RPT = N_PAD // NS
EPW_A = E_PAD // NS
NCHA = EPW_A // K
KB = 48
NCHB = EPW_A // KB
NHALF = N_PAD // 2
ACC_A = 5376

_MESH = plsc.VectorSubcoreMesh(core_axis_name="c", subcore_axis_name="s")


def _mm_body(a_ref, b_ref, o_ref):
    o_ref[...] = jnp.dot(a_ref[...], b_ref[...],
                         preferred_element_type=jnp.float32)


def _matmul(a, b, bm):
    m, k = a.shape
    _, n = b.shape
    return pl.pallas_call(
        _mm_body,
        grid=(m // bm,),
        in_specs=[
            pl.BlockSpec((bm, k), lambda i: (i, 0)),
            pl.BlockSpec((k, n), lambda i: (0, 0)),
        ],
        out_specs=pl.BlockSpec((bm, n), lambda i: (i, 0)),
        out_shape=jax.ShapeDtypeStruct((m, n), jnp.float32),
    )(a, b)


def _mm_acc_body(a_ref, b_ref, o_ref):
    @pl.when(pl.program_id(0) == 0)
    def _():
        o_ref[...] = jnp.zeros_like(o_ref)

    o_ref[...] += jnp.dot(a_ref[...], b_ref[...],
                          preferred_element_type=jnp.float32)


def _matmul_acc(a, b, bk):
    m, k = a.shape
    _, n = b.shape
    return pl.pallas_call(
        _mm_acc_body,
        grid=(k // bk,),
        in_specs=[
            pl.BlockSpec((m, bk), lambda i: (0, i)),
            pl.BlockSpec((bk, n), lambda i: (i, 0)),
        ],
        out_specs=pl.BlockSpec((m, n), lambda i: (0, 0)),
        out_shape=jax.ShapeDtypeStruct((m, n), jnp.float32),
    )(a, b)


def _add_body(a_ref, b_ref, o_ref):
    o_ref[...] = a_ref[...] + b_ref[...]


def _add2d(a, b):
    m, n = a.shape
    return pl.pallas_call(
        _add_body,
        grid=(1,),
        in_specs=[pl.BlockSpec((m, n), lambda i: (0, 0)),
                  pl.BlockSpec((m, n), lambda i: (0, 0))],
        out_specs=pl.BlockSpec((m, n), lambda i: (0, 0)),
        out_shape=jax.ShapeDtypeStruct((m, n), jnp.float32),
    )(a, b)


def _bnstat_body(x_ref, o_ref):
    @pl.when(pl.program_id(0) == 0)
    def _():
        o_ref[...] = jnp.zeros_like(o_ref)

    x = x_ref[...]
    s1 = jnp.sum(x, 0, keepdims=True)
    s2 = jnp.sum(x * x, 0, keepdims=True)
    o_ref[...] += jnp.concatenate(
        [s1, s2, jnp.zeros((6, x.shape[1]), jnp.float32)], 0)


def _bnstat(x, bm):
    m, n = x.shape
    return pl.pallas_call(
        _bnstat_body,
        grid=(m // bm,),
        in_specs=[pl.BlockSpec((bm, n), lambda i: (i, 0))],
        out_specs=pl.BlockSpec((8, n), lambda i: (0, 0)),
        out_shape=jax.ShapeDtypeStruct((8, n), jnp.float32),
    )(x)


def _bnapply_body(agg_ref, h_ref, gb_ref, st_ref, o_ref):
    inv_n = 1.0 / N_NODES
    mean = st_ref[0:1, :] * inv_n
    ex2 = st_ref[1:2, :] * inv_n
    var = ex2 - mean * mean
    inv = lax.rsqrt(var + 1e-5)
    y = (agg_ref[...] - mean) * (inv * gb_ref[0:1, :]) + gb_ref[1:2, :]
    o_ref[...] = h_ref[...] + jnp.maximum(y, 0.0)


def _bnapply(agg, h, gb, stats, bm):
    m, n = agg.shape
    return pl.pallas_call(
        _bnapply_body,
        grid=(m // bm,),
        in_specs=[
            pl.BlockSpec((bm, n), lambda i: (i, 0)),
            pl.BlockSpec((bm, n), lambda i: (i, 0)),
            pl.BlockSpec((8, n), lambda i: (0, 0)),
            pl.BlockSpec((8, n), lambda i: (0, 0)),
        ],
        out_specs=pl.BlockSpec((bm, n), lambda i: (i, 0)),
        out_shape=jax.ShapeDtypeStruct((m, n), jnp.float32),
    )(agg, h, gb, stats)


def _relu_bias_body(x_ref, b_ref, o_ref):
    o_ref[...] = jnp.maximum(x_ref[...] + b_ref[0:1, :], 0.0)


def _relu_bias(x, b, bm):
    m, n = x.shape
    return pl.pallas_call(
        _relu_bias_body,
        grid=(m // bm,),
        in_specs=[pl.BlockSpec((bm, n), lambda i: (i, 0)),
                  pl.BlockSpec((8, n), lambda i: (0, 0))],
        out_specs=pl.BlockSpec((bm, n), lambda i: (i, 0)),
        out_shape=jax.ShapeDtypeStruct((m, n), jnp.float32),
    )(x, b)


def _passA_body(src_h, dst_h, sc_h, se_h, ub_h,
                ex_h, den_h,
                idx_s, idx_d, idx_l, r_s, r_d, r_e, exb, exb16, ubb,
                den_acc, sem):
    c = lax.axis_index("c")
    s = lax.axis_index("s")

    def zrow(k, carry):
        for j in range(8):
            exb[k, pl.ds(j * 16, 16)] = jnp.zeros((16,), jnp.float32)
        return carry
    lax.fori_loop(0, K, zrow, 0)
    for i in range(3):
        j = s + i * NS

        @pl.when(j < ACC_A // K)
        def _():
            pltpu.sync_copy(exb, den_acc.at[pl.ds(j * K, K)])
    plsc.subcore_barrier()

    pltpu.sync_copy(ub_h, ubb)
    ub = ubb[:]
    lo = c * NHALF
    ebase = s * EPW_A

    def chunk(g, carry):
        base = ebase + g * K
        pltpu.sync_copy(src_h.at[pl.ds(base, K)], idx_s)
        pltpu.sync_copy(dst_h.at[pl.ds(base, K)], idx_d)

        def loc(t, cc):
            vl = idx_d[pl.ds(t * 16, 16)] - lo
            ok = (vl >= 0) & (vl < NHALF)
            idx_l[pl.ds(t * 16, 16)] = jnp.where(ok, vl, NHALF)
            return cc
        lax.fori_loop(0, K // 16, loc, 0)

        pltpu.async_copy(sc_h.at[idx_s], r_s, sem).wait()
        pltpu.async_copy(sc_h.at[idx_d], r_d, sem).wait()
        pltpu.sync_copy(se_h.at[pl.ds(base, K)], r_e)

        def ek(k, cc):
            a = (r_s[k, pl.ds(0, 16)] + r_d[k, pl.ds(16, 16)]
                 + r_e[k, pl.ds(0, 16)])
            a = jnp.where(a >= 0.0, a, 0.2 * a)
            ex = jnp.exp(a - ub)
            exb[k, pl.ds(0, 16)] = ex
            exb16[k, :] = ex
            return cc
        lax.fori_loop(0, K, ek, 0)

        @pl.when(c == 0)
        def _():
            pltpu.sync_copy(exb16, ex_h.at[pl.ds(base, K)])

        pltpu.sync_copy(exb, den_acc.at[idx_l], add=True)
        return carry
    lax.fori_loop(0, NCHA, chunk, 0)

    plsc.subcore_barrier()
    rpt = NHALF // NS
    pltpu.sync_copy(den_acc.at[pl.ds(s * rpt, rpt)],
                    den_h.at[pl.ds(c * NHALF + s * rpt, rpt)])


_passA = functools.partial(
    pl.kernel, _passA_body,
    out_type=[
        jax.ShapeDtypeStruct((E_PAD, 16), jnp.float32),
        jax.ShapeDtypeStruct((N_PAD, 128), jnp.float32),
    ],
    mesh=_MESH,
    scratch_types=[
        pltpu.VMEM((K,), jnp.int32),
        pltpu.VMEM((K,), jnp.int32),
        pltpu.VMEM((K,), jnp.int32),
        pltpu.VMEM((K, 128), jnp.float32),
        pltpu.VMEM((K, 128), jnp.float32),
        pltpu.VMEM((K, 128), jnp.float32),
        pltpu.VMEM((K, 128), jnp.float32),
        pltpu.VMEM((K, 16), jnp.float32),
        pltpu.VMEM((16,), jnp.float32),
        pltpu.VMEM_SHARED((ACC_A, 128), jnp.float32),
        pltpu.SemaphoreType.DMA,
    ],
)


_GATHER_DNUMS = lax.GatherDimensionNumbers(
    offset_dims=(), collapsed_slice_dims=(0,), start_index_map=(0,))


def _bcast_lane(v, lane):
    idx = jnp.full((16, 1), lane, jnp.int32)
    return lax.gather(v, idx, _GATHER_DNUMS, slice_sizes=(1,),
                      mode=lax.GatherScatterMode.PROMISE_IN_BOUNDS)


def _passB_body(src_h, dst_h, ex_h, den_h, xh_h,
                out_h,
                idx_s, idx_d, idx_g, rows, exr, denr, msg, acc, sem):
    c = lax.axis_index("c")
    s = lax.axis_index("s")
    wid = s * NC + c

    def zrow(k, carry):
        for j in range(8):
            msg[k, pl.ds(j * 16, 16)] = jnp.zeros((16,), jnp.float32)
        return carry
    lax.fori_loop(0, KB, zrow, 0)
    for off0 in list(range(0, RPT - KB, KB)) + [RPT - KB]:
        pltpu.sync_copy(msg, acc.at[pl.ds(s * RPT + off0, KB)])
    plsc.subcore_barrier()

    ebase = s * EPW_A
    off = c * N_NODES

    def chunk(g, carry):
        base = ebase + g * KB
        pltpu.sync_copy(src_h.at[pl.ds(base, KB)], idx_s)
        pltpu.sync_copy(dst_h.at[pl.ds(base, KB)], idx_d)

        def addoff(t, cc):
            idx_g[pl.ds(t * 16, 16)] = idx_s[pl.ds(t * 16, 16)] + off
            return cc
        lax.fori_loop(0, KB // 16, addoff, 0)

        pltpu.async_copy(xh_h.at[idx_g], rows, sem).wait()
        pltpu.async_copy(den_h.at[idx_d], denr, sem).wait()
        pltpu.sync_copy(ex_h.at[pl.ds(base, KB)], exr)

        def ek(k, cc):
            w16 = exr[k, :] / denr[k, pl.ds(0, 16)] * 0.25
            w0 = _bcast_lane(w16, 0)
            w1 = _bcast_lane(w16, 1)
            w2 = _bcast_lane(w16, 2)
            w3 = _bcast_lane(w16, 3)
            for j in range(8):
                v = w0 * rows[k, pl.ds(j * 16, 16)]
                v += w1 * rows[k, pl.ds(128 + j * 16, 16)]
                v += w2 * rows[k, pl.ds(256 + j * 16, 16)]
                v += w3 * rows[k, pl.ds(384 + j * 16, 16)]
                msg[k, pl.ds(j * 16, 16)] = v
            return cc
        lax.fori_loop(0, KB, ek, 0)

        pltpu.sync_copy(msg, acc.at[idx_d], add=True)
        return carry
    lax.fori_loop(0, NCHB, chunk, 0)

    plsc.subcore_barrier()
    pltpu.sync_copy(acc.at[pl.ds(s * RPT, RPT)],
                    out_h.at[c, pl.ds(s * RPT, RPT)])


_passB = functools.partial(
    pl.kernel, _passB_body,
    out_type=jax.ShapeDtypeStruct((NC, N_PAD, 128), jnp.float32),
    mesh=_MESH,
    scratch_types=[
        pltpu.VMEM((KB,), jnp.int32),
        pltpu.VMEM((KB,), jnp.int32),
        pltpu.VMEM((KB,), jnp.int32),
        pltpu.VMEM((KB, 512), jnp.float32),
        pltpu.VMEM((KB, 16), jnp.float32),
        pltpu.VMEM((KB, 128), jnp.float32),
        pltpu.VMEM((KB, 128), jnp.float32),
        pltpu.VMEM_SHARED((N_PAD, 128), jnp.float32),
        pltpu.SemaphoreType.DMA,
    ],
)


def kernel(x, edge_index, edge_attr, batch, params):
    n = N_NODES
    b_graphs = 64
    f32 = jnp.float32

    loops = jnp.arange(n, dtype=jnp.int32)
    pad_idx = jnp.zeros((E_PAD - E_TOT,), jnp.int32)
    src_full = jnp.concatenate([edge_index[0], loops, pad_idx])
    dst_full = jnp.concatenate([edge_index[1], loops, pad_idx])
    mean_ea = jnp.mean(edge_attr, 0)

    h = _relu_bias(
        _matmul(x, params['W0'], bm=1000),
        jnp.zeros((8, EMB), f32).at[0].set(params['b0']), bm=1000)

    outs = []
    for lp in params['layers']:
        w3 = lp['W'].reshape(EMB, HEADS, EMB)
        w_src = jnp.einsum('khe,he->kh', w3, lp['att_src'])
        w_dst = jnp.einsum('khe,he->kh', w3, lp['att_dst'])
        we3 = lp['We'].reshape(-1, HEADS, EMB)
        we_att = jnp.einsum('khe,he->kh', we3, lp['att_e'])

        wsc = jnp.zeros((EMB, 128), f32)
        wsc = wsc.at[:, 0:HEADS].set(w_src).at[:, 16:16 + HEADS].set(w_dst)
        we_pad = jnp.zeros((16, 128), f32).at[:, 0:HEADS].set(we_att)

        w4 = w3.reshape(EMB, HEADS, 2, 128)
        w_c0 = w4[:, :, 0, :].reshape(EMB, 512)
        w_c1 = w4[:, :, 1, :].reshape(EMB, 512)

        scores = _matmul(h, wsc, bm=1000)
        se_real = _matmul(edge_attr, we_pad, bm=1000)
        se_loop = mean_ea @ we_pad
        se_full = jnp.concatenate([
            se_real,
            jnp.broadcast_to(se_loop, (n, 128)),
            jnp.full((E_PAD - E_TOT, 128), -1e30, f32),
        ], 0)

        ub = jax.nn.leaky_relu(
            jnp.max(scores[:, 0:16], 0) + jnp.max(scores[:, 16:32], 0)
            + jnp.maximum(jnp.max(se_real[:, 0:16], 0), se_loop[0:16]), 0.2)

        ex, den = _passA()(src_full, dst_full, scores, se_full, ub)

        xh0 = _matmul(h, w_c0, bm=1000)
        xh1 = _matmul(h, w_c1, bm=1000)
        xh_cat = jnp.concatenate([xh0, xh1], 0)

        out_t = _passB()(src_full, dst_full, ex, den, xh_cat)
        agg = jnp.concatenate([out_t[0, :n], out_t[1, :n]], axis=1)

        gb = jnp.zeros((8, EMB), f32).at[0].set(lp['gamma']).at[1].set(
            lp['beta'])
        stats = _bnstat(agg, bm=1000)
        h = _bnapply(agg, h, gb, stats, bm=1000)
        outs.append(h)

    batch_pad = jnp.concatenate(
        [batch, jnp.full((N_PAD - n,), -1, batch.dtype)])
    onehot_t = (jnp.arange(b_graphs, dtype=jnp.int32)[:, None]
                == batch_pad[None, :]).astype(f32)
    outs_cat = jnp.concatenate(
        [jnp.concatenate(outs, axis=1),
         jnp.zeros((N_PAD - n, 3 * EMB), f32)], 0)
    pooled_cat = _matmul_acc(onehot_t, outs_cat, bk=2048)

    wg_pad = jnp.zeros((LAYERS_DIM := 3 * EMB, 128), f32).at[:, 0:3].set(
        params['Wg'])
    glin = _matmul(pooled_cat, wg_pad, bm=64)[:, 0:3] + params['bg']
    gates = jax.nn.softmax(glin, axis=1)
    zt = pooled_cat.reshape(b_graphs, 3, EMB)
    z = jnp.sum(zt * gates[..., None], axis=1)
    return (z, outs[-1])


# speedup vs baseline: 8.2853x; 1.0473x over previous
import functools

import jax
import jax.numpy as jnp
from jax import lax
from jax.experimental import pallas as pl
from jax.experimental.pallas import tpu as pltpu
from jax.experimental.pallas import tpu_sc as plsc

HEADS = 4
EMB = 256
N_NODES = 10000
N_PAD = 10240
E_REAL = 160000
E_TOT = 170000
E_PAD = 172032
EPW = 5376
K = 128
NCHUNK = EPW // K
NC = 2
NS = 16
RPT = N_PAD // NS
EPW_A = E_PAD // NS
NCHA = EPW_A // K
KB = 32
NCHB = EPW_A // KB
NHALF = N_PAD // 2
ACC_A = 5248

_MESH = plsc.VectorSubcoreMesh(core_axis_name="c", subcore_axis_name="s")


def _mm_body(a_ref, b_ref, o_ref):
    o_ref[...] = jnp.dot(a_ref[...], b_ref[...],
                         preferred_element_type=jnp.float32)


def _matmul(a, b, bm):
    m, k = a.shape
    _, n = b.shape
    return pl.pallas_call(
        _mm_body,
        grid=(m // bm,),
        in_specs=[
            pl.BlockSpec((bm, k), lambda i: (i, 0)),
            pl.BlockSpec((k, n), lambda i: (0, 0)),
        ],
        out_specs=pl.BlockSpec((bm, n), lambda i: (i, 0)),
        out_shape=jax.ShapeDtypeStruct((m, n), jnp.float32),
    )(a, b)


def _mm_acc_body(a_ref, b_ref, o_ref):
    @pl.when(pl.program_id(0) == 0)
    def _():
        o_ref[...] = jnp.zeros_like(o_ref)

    o_ref[...] += jnp.dot(a_ref[...], b_ref[...],
                          preferred_element_type=jnp.float32)


def _matmul_acc(a, b, bk):
    m, k = a.shape
    _, n = b.shape
    return pl.pallas_call(
        _mm_acc_body,
        grid=(k // bk,),
        in_specs=[
            pl.BlockSpec((m, bk), lambda i: (0, i)),
            pl.BlockSpec((bk, n), lambda i: (i, 0)),
        ],
        out_specs=pl.BlockSpec((m, n), lambda i: (0, 0)),
        out_shape=jax.ShapeDtypeStruct((m, n), jnp.float32),
    )(a, b)


def _add_body(a_ref, b_ref, o_ref):
    o_ref[...] = a_ref[...] + b_ref[...]


def _add2d(a, b):
    m, n = a.shape
    return pl.pallas_call(
        _add_body,
        grid=(1,),
        in_specs=[pl.BlockSpec((m, n), lambda i: (0, 0)),
                  pl.BlockSpec((m, n), lambda i: (0, 0))],
        out_specs=pl.BlockSpec((m, n), lambda i: (0, 0)),
        out_shape=jax.ShapeDtypeStruct((m, n), jnp.float32),
    )(a, b)


def _bnstat_body(x_ref, o_ref):
    @pl.when(pl.program_id(0) == 0)
    def _():
        o_ref[...] = jnp.zeros_like(o_ref)

    x = x_ref[...]
    s1 = jnp.sum(x, 0, keepdims=True)
    s2 = jnp.sum(x * x, 0, keepdims=True)
    o_ref[...] += jnp.concatenate(
        [s1, s2, jnp.zeros((6, x.shape[1]), jnp.float32)], 0)


def _bnstat(x, bm):
    m, n = x.shape
    return pl.pallas_call(
        _bnstat_body,
        grid=(m // bm,),
        in_specs=[pl.BlockSpec((bm, n), lambda i: (i, 0))],
        out_specs=pl.BlockSpec((8, n), lambda i: (0, 0)),
        out_shape=jax.ShapeDtypeStruct((8, n), jnp.float32),
    )(x)


def _bnapply_body(agg_ref, h_ref, gb_ref, st_ref, o_ref):
    inv_n = 1.0 / N_NODES
    mean = st_ref[0:1, :] * inv_n
    ex2 = st_ref[1:2, :] * inv_n
    var = ex2 - mean * mean
    inv = lax.rsqrt(var + 1e-5)
    y = (agg_ref[...] - mean) * (inv * gb_ref[0:1, :]) + gb_ref[1:2, :]
    o_ref[...] = h_ref[...] + jnp.maximum(y, 0.0)


def _bnapply(agg, h, gb, stats, bm):
    m, n = agg.shape
    return pl.pallas_call(
        _bnapply_body,
        grid=(m // bm,),
        in_specs=[
            pl.BlockSpec((bm, n), lambda i: (i, 0)),
            pl.BlockSpec((bm, n), lambda i: (i, 0)),
            pl.BlockSpec((8, n), lambda i: (0, 0)),
            pl.BlockSpec((8, n), lambda i: (0, 0)),
        ],
        out_specs=pl.BlockSpec((bm, n), lambda i: (i, 0)),
        out_shape=jax.ShapeDtypeStruct((m, n), jnp.float32),
    )(agg, h, gb, stats)


def _relu_bias_body(x_ref, b_ref, o_ref):
    o_ref[...] = jnp.maximum(x_ref[...] + b_ref[0:1, :], 0.0)


def _relu_bias(x, b, bm):
    m, n = x.shape
    return pl.pallas_call(
        _relu_bias_body,
        grid=(m // bm,),
        in_specs=[pl.BlockSpec((bm, n), lambda i: (i, 0)),
                  pl.BlockSpec((8, n), lambda i: (0, 0))],
        out_specs=pl.BlockSpec((bm, n), lambda i: (i, 0)),
        out_shape=jax.ShapeDtypeStruct((m, n), jnp.float32),
    )(x, b)


def _passA_body(src_h, dst_h, sc_h, se_h, ub_h,
                ex_h, den_h,
                idx_s, idx_d, idx_l, r_s, r_d, r_e, exb, exb16, ubb,
                den_acc, sem):
    c = lax.axis_index("c")
    s = lax.axis_index("s")

    def zrow(k, carry):
        for j in range(8):
            exb[k, pl.ds(j * 16, 16)] = jnp.zeros((16,), jnp.float32)
        return carry
    lax.fori_loop(0, K, zrow, 0)
    for i in range(3):
        j = s + i * NS

        @pl.when(j < ACC_A // K)
        def _():
            pltpu.sync_copy(exb, den_acc.at[pl.ds(j * K, K)])
    plsc.subcore_barrier()

    pltpu.sync_copy(ub_h, ubb)
    ub = ubb[:]
    lo = c * NHALF
    ebase = s * EPW_A

    def chunk(g, carry):
        base = ebase + g * K
        pltpu.sync_copy(src_h.at[pl.ds(base, K)], idx_s)
        pltpu.sync_copy(dst_h.at[pl.ds(base, K)], idx_d)

        def loc(t, cc):
            vl = idx_d[pl.ds(t * 16, 16)] - lo
            ok = (vl >= 0) & (vl < NHALF)
            idx_l[pl.ds(t * 16, 16)] = jnp.where(ok, vl, NHALF)
            return cc
        lax.fori_loop(0, K // 16, loc, 0)

        pltpu.async_copy(sc_h.at[idx_s], r_s, sem).wait()
        pltpu.async_copy(sc_h.at[idx_d], r_d, sem).wait()
        pltpu.sync_copy(se_h.at[pl.ds(base, K)], r_e)

        def ek(k, cc):
            a = (r_s[k, pl.ds(0, 16)] + r_d[k, pl.ds(16, 16)]
                 + r_e[k, pl.ds(0, 16)])
            a = jnp.where(a >= 0.0, a, 0.2 * a)
            ex = jnp.exp(a - ub)
            exb[k, pl.ds(0, 16)] = ex
            exb16[k, :] = ex
            return cc
        lax.fori_loop(0, K, ek, 0)

        @pl.when(c == 0)
        def _():
            pltpu.sync_copy(exb16, ex_h.at[pl.ds(base, K)])

        pltpu.sync_copy(exb, den_acc.at[idx_l], add=True)
        return carry
    lax.fori_loop(0, NCHA, chunk, 0)

    plsc.subcore_barrier()
    rpt = NHALF // NS
    pltpu.sync_copy(den_acc.at[pl.ds(s * rpt, rpt)],
                    den_h.at[pl.ds(c * NHALF + s * rpt, rpt)])


_passA = functools.partial(
    pl.kernel, _passA_body,
    out_type=[
        jax.ShapeDtypeStruct((E_PAD, 16), jnp.float32),
        jax.ShapeDtypeStruct((N_PAD, 128), jnp.float32),
    ],
    mesh=_MESH,
    scratch_types=[
        pltpu.VMEM((K,), jnp.int32),
        pltpu.VMEM((K,), jnp.int32),
        pltpu.VMEM((K,), jnp.int32),
        pltpu.VMEM((K, 128), jnp.float32),
        pltpu.VMEM((K, 128), jnp.float32),
        pltpu.VMEM((K, 128), jnp.float32),
        pltpu.VMEM((K, 128), jnp.float32),
        pltpu.VMEM((K, 16), jnp.float32),
        pltpu.VMEM((16,), jnp.float32),
        pltpu.VMEM_SHARED((ACC_A, 128), jnp.float32),
        pltpu.SemaphoreType.DMA,
    ],
)


def _passN_body(dst_h, ex_h, den_h, w_h, idx_d, exb, denr, sem):
    c = lax.axis_index("c")
    s = lax.axis_index("s")
    wid = s * NC + c
    ebase = wid * EPW

    def chunk(g, carry):
        base = ebase + g * K
        pltpu.sync_copy(dst_h.at[pl.ds(base, K)], idx_d)
        pltpu.async_copy(den_h.at[idx_d], denr, sem).wait()
        pltpu.sync_copy(ex_h.at[pl.ds(base, K)], exb)

        def wk(k, cc):
            exb[k, :] = exb[k, :] / denr[k, pl.ds(0, 16)] * 0.25
            return cc
        lax.fori_loop(0, K, wk, 0)
        pltpu.sync_copy(exb, w_h.at[pl.ds(base, K)])
        return carry
    lax.fori_loop(0, NCHUNK, chunk, 0)


_passN = functools.partial(
    pl.kernel, _passN_body,
    out_type=jax.ShapeDtypeStruct((E_PAD, 16), jnp.float32),
    mesh=_MESH,
    scratch_types=[
        pltpu.VMEM((K,), jnp.int32),
        pltpu.VMEM((K, 16), jnp.float32),
        pltpu.VMEM((K, 128), jnp.float32),
        pltpu.SemaphoreType.DMA,
    ],
)


_GATHER_DNUMS = lax.GatherDimensionNumbers(
    offset_dims=(), collapsed_slice_dims=(0,), start_index_map=(0,))


def _bcast_lane(v, lane):
    idx = jnp.full((16, 1), lane, jnp.int32)
    return lax.gather(v, idx, _GATHER_DNUMS, slice_sizes=(1,),
                      mode=lax.GatherScatterMode.PROMISE_IN_BOUNDS)


def _passB_body(src_h, dst_h, w_h, xh_h,
                out_h,
                idx_s0, idx_d0, idx_g0, rows0,
                idx_s1, idx_d1, idx_g1, rows1,
                wbuf, msg, acc, sem_r0, sem_r1):
    c = lax.axis_index("c")
    s = lax.axis_index("s")

    def zrow(k, carry):
        for j in range(8):
            msg[k, pl.ds(j * 16, 16)] = jnp.zeros((16,), jnp.float32)
        return carry
    lax.fori_loop(0, KB, zrow, 0)
    for off0 in range(0, RPT, KB):
        pltpu.sync_copy(msg, acc.at[pl.ds(s * RPT + off0, KB)])
    plsc.subcore_barrier()

    ebase = s * EPW_A
    off = c * N_NODES
    bufs = ((idx_s0, idx_d0, idx_g0, rows0, sem_r0),
            (idx_s1, idx_d1, idx_g1, rows1, sem_r1))

    def fire(g, p):
        idx_s, idx_d, idx_g, rows, sem_r = bufs[p]
        base = ebase + g * KB
        pltpu.sync_copy(src_h.at[pl.ds(base, KB)], idx_s)
        pltpu.sync_copy(dst_h.at[pl.ds(base, KB)], idx_d)

        def addoff(t, cc):
            idx_g[pl.ds(t * 16, 16)] = idx_s[pl.ds(t * 16, 16)] + off
            return cc
        lax.fori_loop(0, KB // 16, addoff, 0)
        pltpu.async_copy(xh_h.at[idx_g], rows, sem_r)

    def process(g, p):
        idx_s, idx_d, idx_g, rows, sem_r = bufs[p]
        base = ebase + g * KB
        pltpu.make_async_copy(xh_h.at[idx_g], rows, sem_r).wait()
        pltpu.sync_copy(w_h.at[pl.ds(base, KB)], wbuf)

        def ek(k, cc):
            w16 = wbuf[k, :]
            w0 = _bcast_lane(w16, 0)
            w1 = _bcast_lane(w16, 1)
            w2 = _bcast_lane(w16, 2)
            w3 = _bcast_lane(w16, 3)
            for j in range(8):
                v = w0 * rows[k, pl.ds(j * 16, 16)]
                v += w1 * rows[k, pl.ds(128 + j * 16, 16)]
                v += w2 * rows[k, pl.ds(256 + j * 16, 16)]
                v += w3 * rows[k, pl.ds(384 + j * 16, 16)]
                msg[k, pl.ds(j * 16, 16)] = v
            return cc
        lax.fori_loop(0, KB, ek, 0)

        pltpu.sync_copy(msg, acc.at[idx_d], add=True)

    fire(0, 0)

    def pair(i, carry):
        g0 = i * 2
        fire(g0 + 1, 1)
        process(g0, 0)

        @pl.when(g0 + 2 < NCHB)
        def _():
            fire(g0 + 2, 0)
        process(g0 + 1, 1)
        return carry
    lax.fori_loop(0, NCHB // 2, pair, 0)

    plsc.subcore_barrier()
    pltpu.sync_copy(acc.at[pl.ds(s * RPT, RPT)],
                    out_h.at[c, pl.ds(s * RPT, RPT)])


_passB = functools.partial(
    pl.kernel, _passB_body,
    out_type=jax.ShapeDtypeStruct((NC, N_PAD, 128), jnp.float32),
    mesh=_MESH,
    scratch_types=[
        pltpu.VMEM((KB,), jnp.int32),
        pltpu.VMEM((KB,), jnp.int32),
        pltpu.VMEM((KB,), jnp.int32),
        pltpu.VMEM((KB, 512), jnp.float32),
        pltpu.VMEM((KB,), jnp.int32),
        pltpu.VMEM((KB,), jnp.int32),
        pltpu.VMEM((KB,), jnp.int32),
        pltpu.VMEM((KB, 512), jnp.float32),
        pltpu.VMEM((KB, 16), jnp.float32),
        pltpu.VMEM((KB, 128), jnp.float32),
        pltpu.VMEM_SHARED((N_PAD, 128), jnp.float32),
        pltpu.SemaphoreType.DMA,
        pltpu.SemaphoreType.DMA,
    ],
)


def kernel(x, edge_index, edge_attr, batch, params):
    n = N_NODES
    b_graphs = 64
    f32 = jnp.float32

    loops = jnp.arange(n, dtype=jnp.int32)
    pad_idx = jnp.zeros((E_PAD - E_TOT,), jnp.int32)
    src_full = jnp.concatenate([edge_index[0], loops, pad_idx])
    dst_full = jnp.concatenate([edge_index[1], loops, pad_idx])
    mean_ea = jnp.mean(edge_attr, 0)

    h = _relu_bias(
        _matmul(x, params['W0'], bm=1000),
        jnp.zeros((8, EMB), f32).at[0].set(params['b0']), bm=1000)

    outs = []
    for lp in params['layers']:
        w3 = lp['W'].reshape(EMB, HEADS, EMB)
        w_src = jnp.einsum('khe,he->kh', w3, lp['att_src'])
        w_dst = jnp.einsum('khe,he->kh', w3, lp['att_dst'])
        we3 = lp['We'].reshape(-1, HEADS, EMB)
        we_att = jnp.einsum('khe,he->kh', we3, lp['att_e'])

        wsc = jnp.zeros((EMB, 128), f32)
        wsc = wsc.at[:, 0:HEADS].set(w_src).at[:, 16:16 + HEADS].set(w_dst)
        we_pad = jnp.zeros((16, 128), f32).at[:, 0:HEADS].set(we_att)

        w4 = w3.reshape(EMB, HEADS, 2, 128)
        w_c0 = w4[:, :, 0, :].reshape(EMB, 512)
        w_c1 = w4[:, :, 1, :].reshape(EMB, 512)

        scores = _matmul(h, wsc, bm=1000)
        se_real = _matmul(edge_attr, we_pad, bm=1000)
        se_loop = mean_ea @ we_pad
        se_full = jnp.concatenate([
            se_real,
            jnp.broadcast_to(se_loop, (n, 128)),
            jnp.full((E_PAD - E_TOT, 128), -1e30, f32),
        ], 0)

        ub = jax.nn.leaky_relu(
            jnp.max(scores[:, 0:16], 0) + jnp.max(scores[:, 16:32], 0)
            + jnp.maximum(jnp.max(se_real[:, 0:16], 0), se_loop[0:16]), 0.2)

        ex, den = _passA()(src_full, dst_full, scores, se_full, ub)
        w_edge = _passN()(dst_full, ex, den)

        xh0 = _matmul(h, w_c0, bm=1000)
        xh1 = _matmul(h, w_c1, bm=1000)
        xh_cat = jnp.concatenate([xh0, xh1], 0)

        out_t = _passB()(src_full, dst_full, w_edge, xh_cat)
        agg = jnp.concatenate([out_t[0, :n], out_t[1, :n]], axis=1)

        gb = jnp.zeros((8, EMB), f32).at[0].set(lp['gamma']).at[1].set(
            lp['beta'])
        stats = _bnstat(agg, bm=1000)
        h = _bnapply(agg, h, gb, stats, bm=1000)
        outs.append(h)

    batch_pad = jnp.concatenate(
        [batch, jnp.full((N_PAD - n,), -1, batch.dtype)])
    onehot_t = (jnp.arange(b_graphs, dtype=jnp.int32)[:, None]
                == batch_pad[None, :]).astype(f32)
    outs_cat = jnp.concatenate(
        [jnp.concatenate(outs, axis=1),
         jnp.zeros((N_PAD - n, 3 * EMB), f32)], 0)
    pooled_cat = _matmul_acc(onehot_t, outs_cat, bk=2048)

    wg_pad = jnp.zeros((LAYERS_DIM := 3 * EMB, 128), f32).at[:, 0:3].set(
        params['Wg'])
    glin = _matmul(pooled_cat, wg_pad, bm=64)[:, 0:3] + params['bg']
    gates = jax.nn.softmax(glin, axis=1)
    zt = pooled_cat.reshape(b_graphs, 3, EMB)
    z = jnp.sum(zt * gates[..., None], axis=1)
    return (z, outs[-1])


# speedup vs baseline: 8.9150x; 1.0760x over previous
import functools

import jax
import jax.numpy as jnp
from jax import lax
from jax.experimental import pallas as pl
from jax.experimental.pallas import tpu as pltpu
from jax.experimental.pallas import tpu_sc as plsc

HEADS = 4
EMB = 256
N_NODES = 10000
N_PAD = 10240
E_REAL = 160000
E_TOT = 170000
E_PAD = 172032
EPW = 5376
K = 128
NCHUNK = EPW // K
NC = 2
NS = 16
RPT = N_PAD // NS
EPW_A = E_PAD // NS
KA = 64
NCHA = EPW_A // KA
KB = 32
NCHB = EPW_A // KB
NHALF = N_PAD // 2
ACC_A = 5248

_MESH = plsc.VectorSubcoreMesh(core_axis_name="c", subcore_axis_name="s")


def _mm_body(a_ref, b_ref, o_ref):
    o_ref[...] = jnp.dot(a_ref[...], b_ref[...],
                         preferred_element_type=jnp.float32)


def _matmul(a, b, bm):
    m, k = a.shape
    _, n = b.shape
    return pl.pallas_call(
        _mm_body,
        grid=(m // bm,),
        in_specs=[
            pl.BlockSpec((bm, k), lambda i: (i, 0)),
            pl.BlockSpec((k, n), lambda i: (0, 0)),
        ],
        out_specs=pl.BlockSpec((bm, n), lambda i: (i, 0)),
        out_shape=jax.ShapeDtypeStruct((m, n), jnp.float32),
    )(a, b)


def _mm_acc_body(a_ref, b_ref, o_ref):
    @pl.when(pl.program_id(0) == 0)
    def _():
        o_ref[...] = jnp.zeros_like(o_ref)

    o_ref[...] += jnp.dot(a_ref[...], b_ref[...],
                          preferred_element_type=jnp.float32)


def _matmul_acc(a, b, bk):
    m, k = a.shape
    _, n = b.shape
    return pl.pallas_call(
        _mm_acc_body,
        grid=(k // bk,),
        in_specs=[
            pl.BlockSpec((m, bk), lambda i: (0, i)),
            pl.BlockSpec((bk, n), lambda i: (i, 0)),
        ],
        out_specs=pl.BlockSpec((m, n), lambda i: (0, 0)),
        out_shape=jax.ShapeDtypeStruct((m, n), jnp.float32),
    )(a, b)


def _add_body(a_ref, b_ref, o_ref):
    o_ref[...] = a_ref[...] + b_ref[...]


def _add2d(a, b):
    m, n = a.shape
    return pl.pallas_call(
        _add_body,
        grid=(1,),
        in_specs=[pl.BlockSpec((m, n), lambda i: (0, 0)),
                  pl.BlockSpec((m, n), lambda i: (0, 0))],
        out_specs=pl.BlockSpec((m, n), lambda i: (0, 0)),
        out_shape=jax.ShapeDtypeStruct((m, n), jnp.float32),
    )(a, b)


def _bnstat_body(x_ref, o_ref):
    @pl.when(pl.program_id(0) == 0)
    def _():
        o_ref[...] = jnp.zeros_like(o_ref)

    x = x_ref[...]
    s1 = jnp.sum(x, 0, keepdims=True)
    s2 = jnp.sum(x * x, 0, keepdims=True)
    o_ref[...] += jnp.concatenate(
        [s1, s2, jnp.zeros((6, x.shape[1]), jnp.float32)], 0)


def _bnstat(x, bm):
    m, n = x.shape
    return pl.pallas_call(
        _bnstat_body,
        grid=(m // bm,),
        in_specs=[pl.BlockSpec((bm, n), lambda i: (i, 0))],
        out_specs=pl.BlockSpec((8, n), lambda i: (0, 0)),
        out_shape=jax.ShapeDtypeStruct((8, n), jnp.float32),
    )(x)


def _bnapply_body(agg_ref, h_ref, gb_ref, st_ref, o_ref):
    inv_n = 1.0 / N_NODES
    mean = st_ref[0:1, :] * inv_n
    ex2 = st_ref[1:2, :] * inv_n
    var = ex2 - mean * mean
    inv = lax.rsqrt(var + 1e-5)
    y = (agg_ref[...] - mean) * (inv * gb_ref[0:1, :]) + gb_ref[1:2, :]
    o_ref[...] = h_ref[...] + jnp.maximum(y, 0.0)


def _bnapply(agg, h, gb, stats, bm):
    m, n = agg.shape
    return pl.pallas_call(
        _bnapply_body,
        grid=(m // bm,),
        in_specs=[
            pl.BlockSpec((bm, n), lambda i: (i, 0)),
            pl.BlockSpec((bm, n), lambda i: (i, 0)),
            pl.BlockSpec((8, n), lambda i: (0, 0)),
            pl.BlockSpec((8, n), lambda i: (0, 0)),
        ],
        out_specs=pl.BlockSpec((bm, n), lambda i: (i, 0)),
        out_shape=jax.ShapeDtypeStruct((m, n), jnp.float32),
    )(agg, h, gb, stats)


def _relu_bias_body(x_ref, b_ref, o_ref):
    o_ref[...] = jnp.maximum(x_ref[...] + b_ref[0:1, :], 0.0)


def _relu_bias(x, b, bm):
    m, n = x.shape
    return pl.pallas_call(
        _relu_bias_body,
        grid=(m // bm,),
        in_specs=[pl.BlockSpec((bm, n), lambda i: (i, 0)),
                  pl.BlockSpec((8, n), lambda i: (0, 0))],
        out_specs=pl.BlockSpec((bm, n), lambda i: (i, 0)),
        out_shape=jax.ShapeDtypeStruct((m, n), jnp.float32),
    )(x, b)


def _passA_body(src_h, dst_h, sc_h, se_h, ub_h,
                ex_h, den_h,
                idx_s0, idx_d0, idx_l0, r_s0, r_d0, r_e0,
                idx_s1, idx_d1, idx_l1, r_s1, r_d1, r_e1,
                exb, exb16, ubb, den_acc, sem0, sem1):
    c = lax.axis_index("c")
    s = lax.axis_index("s")

    def zrow(k, carry):
        for j in range(8):
            exb[k, pl.ds(j * 16, 16)] = jnp.zeros((16,), jnp.float32)
        return carry
    lax.fori_loop(0, KA, zrow, 0)
    for i in range(ACC_A // KA // NS + 1):
        j = s + i * NS

        @pl.when(j < ACC_A // KA)
        def _():
            pltpu.sync_copy(exb, den_acc.at[pl.ds(j * KA, KA)])
    plsc.subcore_barrier()

    pltpu.sync_copy(ub_h, ubb)
    ub = ubb[:]
    lo = c * NHALF
    ebase = s * EPW_A
    bufs = ((idx_s0, idx_d0, idx_l0, r_s0, r_d0, r_e0, sem0),
            (idx_s1, idx_d1, idx_l1, r_s1, r_d1, r_e1, sem1))

    def fire(g, p):
        idx_s, idx_d, idx_l, r_s, r_d, r_e, sem = bufs[p]
        base = ebase + g * KA
        pltpu.sync_copy(src_h.at[pl.ds(base, KA)], idx_s)
        pltpu.sync_copy(dst_h.at[pl.ds(base, KA)], idx_d)

        def loc(t, cc):
            vl = idx_d[pl.ds(t * 16, 16)] - lo
            ok = (vl >= 0) & (vl < NHALF)
            idx_l[pl.ds(t * 16, 16)] = jnp.where(ok, vl, NHALF)
            return cc
        lax.fori_loop(0, KA // 16, loc, 0)

        pltpu.async_copy(sc_h.at[idx_s], r_s, sem)
        pltpu.async_copy(sc_h.at[idx_d], r_d, sem)
        pltpu.async_copy(se_h.at[pl.ds(base, KA)], r_e, sem)

    def process(g, p):
        idx_s, idx_d, idx_l, r_s, r_d, r_e, sem = bufs[p]
        base = ebase + g * KA
        pltpu.make_async_copy(sc_h.at[idx_s], r_s, sem).wait()
        pltpu.make_async_copy(sc_h.at[idx_d], r_d, sem).wait()
        pltpu.make_async_copy(se_h.at[pl.ds(base, KA)], r_e, sem).wait()

        def ek(k, cc):
            a = (r_s[k, pl.ds(0, 16)] + r_d[k, pl.ds(16, 16)]
                 + r_e[k, pl.ds(0, 16)])
            a = jnp.where(a >= 0.0, a, 0.2 * a)
            ex = jnp.exp(a - ub)
            exb[k, pl.ds(0, 16)] = ex
            exb16[k, :] = ex
            return cc
        lax.fori_loop(0, KA, ek, 0)

        @pl.when(c == 0)
        def _():
            pltpu.sync_copy(exb16, ex_h.at[pl.ds(base, KA)])

        pltpu.sync_copy(exb, den_acc.at[idx_l], add=True)

    fire(0, 0)

    def pair(i, carry):
        g0 = i * 2
        fire(g0 + 1, 1)
        process(g0, 0)

        @pl.when(g0 + 2 < NCHA)
        def _():
            fire(g0 + 2, 0)
        process(g0 + 1, 1)
        return carry
    lax.fori_loop(0, NCHA // 2, pair, 0)

    plsc.subcore_barrier()
    rpt = NHALF // NS
    pltpu.sync_copy(den_acc.at[pl.ds(s * rpt, rpt)],
                    den_h.at[pl.ds(c * NHALF + s * rpt, rpt)])


_passA = functools.partial(
    pl.kernel, _passA_body,
    out_type=[
        jax.ShapeDtypeStruct((E_PAD, 16), jnp.float32),
        jax.ShapeDtypeStruct((N_PAD, 128), jnp.float32),
    ],
    mesh=_MESH,
    scratch_types=[
        pltpu.VMEM((KA,), jnp.int32),
        pltpu.VMEM((KA,), jnp.int32),
        pltpu.VMEM((KA,), jnp.int32),
        pltpu.VMEM((KA, 128), jnp.float32),
        pltpu.VMEM((KA, 128), jnp.float32),
        pltpu.VMEM((KA, 128), jnp.float32),
        pltpu.VMEM((KA,), jnp.int32),
        pltpu.VMEM((KA,), jnp.int32),
        pltpu.VMEM((KA,), jnp.int32),
        pltpu.VMEM((KA, 128), jnp.float32),
        pltpu.VMEM((KA, 128), jnp.float32),
        pltpu.VMEM((KA, 128), jnp.float32),
        pltpu.VMEM((KA, 128), jnp.float32),
        pltpu.VMEM((KA, 16), jnp.float32),
        pltpu.VMEM((16,), jnp.float32),
        pltpu.VMEM_SHARED((ACC_A, 128), jnp.float32),
        pltpu.SemaphoreType.DMA,
        pltpu.SemaphoreType.DMA,
    ],
)


def _passN_body(dst_h, ex_h, den_h, w_h, idx_d, exb, denr, sem):
    c = lax.axis_index("c")
    s = lax.axis_index("s")
    wid = s * NC + c
    ebase = wid * EPW

    def chunk(g, carry):
        base = ebase + g * K
        pltpu.sync_copy(dst_h.at[pl.ds(base, K)], idx_d)
        pltpu.async_copy(den_h.at[idx_d], denr, sem).wait()
        pltpu.sync_copy(ex_h.at[pl.ds(base, K)], exb)

        def wk(k, cc):
            exb[k, :] = exb[k, :] / denr[k, pl.ds(0, 16)] * 0.25
            return cc
        lax.fori_loop(0, K, wk, 0)
        pltpu.sync_copy(exb, w_h.at[pl.ds(base, K)])
        return carry
    lax.fori_loop(0, NCHUNK, chunk, 0)


_passN = functools.partial(
    pl.kernel, _passN_body,
    out_type=jax.ShapeDtypeStruct((E_PAD, 16), jnp.float32),
    mesh=_MESH,
    scratch_types=[
        pltpu.VMEM((K,), jnp.int32),
        pltpu.VMEM((K, 16), jnp.float32),
        pltpu.VMEM((K, 128), jnp.float32),
        pltpu.SemaphoreType.DMA,
    ],
)


_GATHER_DNUMS = lax.GatherDimensionNumbers(
    offset_dims=(), collapsed_slice_dims=(0,), start_index_map=(0,))


def _bcast_lane(v, lane):
    idx = jnp.full((16, 1), lane, jnp.int32)
    return lax.gather(v, idx, _GATHER_DNUMS, slice_sizes=(1,),
                      mode=lax.GatherScatterMode.PROMISE_IN_BOUNDS)


def _passB_body(src_h, dst_h, w_h, xh_h,
                out_h,
                idx_s0, idx_d0, idx_g0, rows0,
                idx_s1, idx_d1, idx_g1, rows1,
                wbuf, msg, acc, sem_r0, sem_r1):
    c = lax.axis_index("c")
    s = lax.axis_index("s")

    def zrow(k, carry):
        for j in range(8):
            msg[k, pl.ds(j * 16, 16)] = jnp.zeros((16,), jnp.float32)
        return carry
    lax.fori_loop(0, KB, zrow, 0)
    for off0 in range(0, RPT, KB):
        pltpu.sync_copy(msg, acc.at[pl.ds(s * RPT + off0, KB)])
    plsc.subcore_barrier()

    ebase = s * EPW_A
    off = c * N_NODES
    bufs = ((idx_s0, idx_d0, idx_g0, rows0, sem_r0),
            (idx_s1, idx_d1, idx_g1, rows1, sem_r1))

    def fire(g, p):
        idx_s, idx_d, idx_g, rows, sem_r = bufs[p]
        base = ebase + g * KB
        pltpu.sync_copy(src_h.at[pl.ds(base, KB)], idx_s)
        pltpu.sync_copy(dst_h.at[pl.ds(base, KB)], idx_d)

        def addoff(t, cc):
            idx_g[pl.ds(t * 16, 16)] = idx_s[pl.ds(t * 16, 16)] + off
            return cc
        lax.fori_loop(0, KB // 16, addoff, 0)
        pltpu.async_copy(xh_h.at[idx_g], rows, sem_r)

    def process(g, p):
        idx_s, idx_d, idx_g, rows, sem_r = bufs[p]
        base = ebase + g * KB
        pltpu.make_async_copy(xh_h.at[idx_g], rows, sem_r).wait()
        pltpu.sync_copy(w_h.at[pl.ds(base, KB)], wbuf)

        def ek(k, cc):
            w16 = wbuf[k, :]
            w0 = _bcast_lane(w16, 0)
            w1 = _bcast_lane(w16, 1)
            w2 = _bcast_lane(w16, 2)
            w3 = _bcast_lane(w16, 3)
            for j in range(8):
                v = w0 * rows[k, pl.ds(j * 16, 16)]
                v += w1 * rows[k, pl.ds(128 + j * 16, 16)]
                v += w2 * rows[k, pl.ds(256 + j * 16, 16)]
                v += w3 * rows[k, pl.ds(384 + j * 16, 16)]
                msg[k, pl.ds(j * 16, 16)] = v
            return cc
        lax.fori_loop(0, KB, ek, 0)

        pltpu.sync_copy(msg, acc.at[idx_d], add=True)

    fire(0, 0)

    def pair(i, carry):
        g0 = i * 2
        fire(g0 + 1, 1)
        process(g0, 0)

        @pl.when(g0 + 2 < NCHB)
        def _():
            fire(g0 + 2, 0)
        process(g0 + 1, 1)
        return carry
    lax.fori_loop(0, NCHB // 2, pair, 0)

    plsc.subcore_barrier()
    pltpu.sync_copy(acc.at[pl.ds(s * RPT, RPT)],
                    out_h.at[c, pl.ds(s * RPT, RPT)])


_passB = functools.partial(
    pl.kernel, _passB_body,
    out_type=jax.ShapeDtypeStruct((NC, N_PAD, 128), jnp.float32),
    mesh=_MESH,
    scratch_types=[
        pltpu.VMEM((KB,), jnp.int32),
        pltpu.VMEM((KB,), jnp.int32),
        pltpu.VMEM((KB,), jnp.int32),
        pltpu.VMEM((KB, 512), jnp.float32),
        pltpu.VMEM((KB,), jnp.int32),
        pltpu.VMEM((KB,), jnp.int32),
        pltpu.VMEM((KB,), jnp.int32),
        pltpu.VMEM((KB, 512), jnp.float32),
        pltpu.VMEM((KB, 16), jnp.float32),
        pltpu.VMEM((KB, 128), jnp.float32),
        pltpu.VMEM_SHARED((N_PAD, 128), jnp.float32),
        pltpu.SemaphoreType.DMA,
        pltpu.SemaphoreType.DMA,
    ],
)


def kernel(x, edge_index, edge_attr, batch, params):
    n = N_NODES
    b_graphs = 64
    f32 = jnp.float32

    loops = jnp.arange(n, dtype=jnp.int32)
    pad_idx = jnp.zeros((E_PAD - E_TOT,), jnp.int32)
    src_full = jnp.concatenate([edge_index[0], loops, pad_idx])
    dst_full = jnp.concatenate([edge_index[1], loops, pad_idx])
    mean_ea = jnp.mean(edge_attr, 0)

    h = _relu_bias(
        _matmul(x, params['W0'], bm=1000),
        jnp.zeros((8, EMB), f32).at[0].set(params['b0']), bm=1000)

    outs = []
    for lp in params['layers']:
        w3 = lp['W'].reshape(EMB, HEADS, EMB)
        w_src = jnp.einsum('khe,he->kh', w3, lp['att_src'])
        w_dst = jnp.einsum('khe,he->kh', w3, lp['att_dst'])
        we3 = lp['We'].reshape(-1, HEADS, EMB)
        we_att = jnp.einsum('khe,he->kh', we3, lp['att_e'])

        wsc = jnp.zeros((EMB, 128), f32)
        wsc = wsc.at[:, 0:HEADS].set(w_src).at[:, 16:16 + HEADS].set(w_dst)
        we_pad = jnp.zeros((16, 128), f32).at[:, 0:HEADS].set(we_att)

        w4 = w3.reshape(EMB, HEADS, 2, 128)
        w_c0 = w4[:, :, 0, :].reshape(EMB, 512)
        w_c1 = w4[:, :, 1, :].reshape(EMB, 512)

        scores = _matmul(h, wsc, bm=1000)
        se_real = _matmul(edge_attr, we_pad, bm=1000)
        se_loop = mean_ea @ we_pad
        se_full = jnp.concatenate([
            se_real,
            jnp.broadcast_to(se_loop, (n, 128)),
            jnp.full((E_PAD - E_TOT, 128), -1e30, f32),
        ], 0)

        ub = jax.nn.leaky_relu(
            jnp.max(scores[:, 0:16], 0) + jnp.max(scores[:, 16:32], 0)
            + jnp.maximum(jnp.max(se_real[:, 0:16], 0), se_loop[0:16]), 0.2)

        ex, den = _passA()(src_full, dst_full, scores, se_full, ub)
        w_edge = _passN()(dst_full, ex, den)

        xh0 = _matmul(h, w_c0, bm=1000)
        xh1 = _matmul(h, w_c1, bm=1000)
        xh_cat = jnp.concatenate([xh0, xh1], 0)

        out_t = _passB()(src_full, dst_full, w_edge, xh_cat)
        agg = jnp.concatenate([out_t[0, :n], out_t[1, :n]], axis=1)

        gb = jnp.zeros((8, EMB), f32).at[0].set(lp['gamma']).at[1].set(
            lp['beta'])
        stats = _bnstat(agg, bm=1000)
        h = _bnapply(agg, h, gb, stats, bm=1000)
        outs.append(h)

    batch_pad = jnp.concatenate(
        [batch, jnp.full((N_PAD - n,), -1, batch.dtype)])
    onehot_t = (jnp.arange(b_graphs, dtype=jnp.int32)[:, None]
                == batch_pad[None, :]).astype(f32)
    outs_cat = jnp.concatenate(
        [jnp.concatenate(outs, axis=1),
         jnp.zeros((N_PAD - n, 3 * EMB), f32)], 0)
    pooled_cat = _matmul_acc(onehot_t, outs_cat, bk=2048)

    wg_pad = jnp.zeros((LAYERS_DIM := 3 * EMB, 128), f32).at[:, 0:3].set(
        params['Wg'])
    glin = _matmul(pooled_cat, wg_pad, bm=64)[:, 0:3] + params['bg']
    gates = jax.nn.softmax(glin, axis=1)
    zt = pooled_cat.reshape(b_graphs, 3, EMB)
    z = jnp.sum(zt * gates[..., None], axis=1)
    return (z, outs[-1])
